# Initial kernel scaffold; baseline (speedup 1.0000x reference)
import jax
import jax.numpy as jnp
from jax.experimental import pallas as pl


def kernel(positions, supernode_indices, W1, b1, W2, b2, Wp, bp):
    raise NotImplementedError("write your pallas kernel here")


# speedup vs baseline: 3.2127x; 3.2127x over previous
import functools

import jax
import jax.numpy as jnp
from jax import lax
from jax.experimental import pallas as pl
from jax.experimental.pallas import tpu as pltpu
from jax.experimental.pallas import tpu_sc as plsc

_HID = 256
_K = 16
_N = 32768
_S = 1024
_PAD_D = 16

_TOPK_BS = 128
_TOPK_C = 2048
_TOPK_CH = 256
_MLP_ROWS = 2048
_MLP_SUP = _MLP_ROWS // _K

_LOG1E4 = 9.210340371976184


def _sc_gather_rows(table, idx):
    B = idx.shape[0]
    info = plsc.get_sparse_core_info()
    nw = info.num_cores * info.num_subcores
    bpw = B // nw
    mesh = plsc.VectorSubcoreMesh(core_axis_name="c", subcore_axis_name="s")

    @functools.partial(
        pl.kernel,
        mesh=mesh,
        compiler_params=pltpu.CompilerParams(use_tc_tiling_on_sc=False),
        out_type=jax.ShapeDtypeStruct((B, _PAD_D), jnp.float32),
        scratch_types=[
            pltpu.VMEM((bpw,), jnp.int32),
            pltpu.VMEM((bpw, _PAD_D), jnp.float32),
            pltpu.SemaphoreType.DMA,
        ],
    )
    def gather_kernel(table_hbm, idx_hbm, out_hbm, idx_v, rows_v, sem):
        wid = lax.axis_index("s") * info.num_cores + lax.axis_index("c")
        base = wid * bpw
        pltpu.sync_copy(idx_hbm.at[pl.ds(base, bpw)], idx_v)
        pltpu.async_copy(table_hbm.at[idx_v], rows_v, sem).wait()
        pltpu.sync_copy(rows_v, out_hbm.at[pl.ds(base, bpw)])

    return gather_kernel(table, idx)


def _topk_body(pos_ref, sup_ref, out_ref, d2_ref):
    n_chunks = _N // _TOPK_C
    s0 = sup_ref[:, 0:1]
    s1 = sup_ref[:, 1:2]
    s2 = sup_ref[:, 2:3]

    bl = lax.broadcasted_iota(jnp.int32, (_TOPK_BS, _TOPK_C), 1)

    def build(c, carry):
        sl = pl.ds(c * _TOPK_C, _TOPK_C)
        p0 = pos_ref[0:1, sl]
        p1 = pos_ref[1:2, sl]
        p2 = pos_ref[2:3, sl]
        t0 = s0 - p0
        t1 = s1 - p1
        t2 = s2 - p2
        d2 = (t0 * t0 + t1 * t1) + t2 * t2
        cid = jax.lax.shift_right_logical(bl, 8) + (c * (_TOPK_C // _TOPK_CH))
        bits = jax.lax.bitcast_convert_type(d2 + 1.0, jnp.int32)
        keys = jax.lax.bitwise_or(jax.lax.bitwise_and(bits, -128), cid)
        d2_ref[:, sl] = jax.lax.bitcast_convert_type(keys, jnp.float32)
        return carry

    lax.fori_loop(0, n_chunks, build, 0)

    lane_k = lax.broadcasted_iota(jnp.int32, (_TOPK_BS, _K), 1)
    big = jnp.float32(jnp.inf)
    ch = _TOPK_CH
    n_ch2 = _N // ch
    lane_ch = lax.broadcasted_iota(jnp.int32, (_TOPK_BS, ch), 1)

    def extract(t, carry):
        pi, acc = carry

        def scan_chunk(c4, av):
            for u in range(8):
                c = c4 * 8 + u
                sl = pl.ds(c * ch, ch)
                dc = d2_ref[:, sl]
                dcm = jnp.where(lane_ch == (pi - c * ch)[:, None], big, dc)
                d2_ref[:, sl] = dcm
                av = jnp.minimum(av, dcm)
            return av

        av0 = jnp.full((_TOPK_BS, ch), big, jnp.float32)
        av = lax.fori_loop(0, n_ch2 // 8, scan_chunk, av0)
        mv = jnp.min(av, axis=1)
        li = jnp.min(jnp.where(av == mv[:, None], lane_ch, _N), axis=1)
        cw = jax.lax.bitwise_and(
            jax.lax.bitcast_convert_type(mv, jnp.int32), 127)
        bi = cw * ch + li
        acc = jnp.where(lane_k == t, bi[:, None], acc)
        return bi, acc

    pi0 = jnp.full((_TOPK_BS,), -1, jnp.int32)
    acc0 = jnp.zeros((_TOPK_BS, _K), jnp.int32)
    _, idxs = lax.fori_loop(0, _K, extract, (pi0, acc0))
    out_ref[...] = idxs


def _topk(pos_t, sup16):
    grid = _S // _TOPK_BS
    return pl.pallas_call(
        _topk_body,
        grid=(grid,),
        in_specs=[
            pl.BlockSpec((3, _N), lambda i: (0, 0)),
            pl.BlockSpec((_TOPK_BS, _PAD_D), lambda i: (i, 0)),
        ],
        out_specs=pl.BlockSpec((_TOPK_BS, _K), lambda i: (i, 0)),
        out_shape=jax.ShapeDtypeStruct((_S, _K), jnp.int32),
        scratch_shapes=[pltpu.VMEM((_TOPK_BS, _N), jnp.float32)],
    )(pos_t, sup16)


def _sincos_block(vals, widths, half):
    R = vals[0].shape[0]
    nb = len(vals)
    fb = jnp.concatenate(
        [jnp.broadcast_to(v, (R, half)) for v in vals], axis=1)
    ji = lax.broadcasted_iota(jnp.int32, (1, nb * half), 1)
    jm = (ji % half).astype(jnp.float32)
    om = 1.0 / jnp.exp((jm / half) * _LOG1E4)
    ang = fb * om
    sn = jnp.sin(ang)
    cs = jnp.cos(ang)
    parts = []
    for c in range(nb):
        parts.append(sn[:, c * half:(c + 1) * half])
        parts.append(cs[:, c * half:(c + 1) * half])
    if widths > 2 * half * nb:
        parts.append(jnp.zeros((R, widths - 2 * half * nb), jnp.float32))
    return jnp.concatenate(parts, axis=1)


def _mlp_body(nbr_ref, sup_ref, w1_ref, b1_ref, w2_ref, b2_ref, wp_ref,
              bp_ref, out_ref):
    R, BSUP = _MLP_ROWS, _MLP_SUP
    hi = lax.Precision.HIGHEST
    nbr = nbr_ref[...]
    sup = sup_ref[...]

    r_iota = lax.broadcasted_iota(jnp.int32, (R, BSUP), 0)
    s_iota = lax.broadcasted_iota(jnp.int32, (R, BSUP), 1)
    expand = (r_iota // _K == s_iota).astype(jnp.float32)
    sup_rows = lax.dot_general(expand, sup, (((1,), (0,)), ((), ())),
                               precision=hi)
    rel = nbr - sup_rows
    rsq = rel * rel
    d2 = (rsq[:, 0:1] + rsq[:, 1:2]) + rsq[:, 2:3]
    d = jnp.sqrt(d2 + 1e-12)

    emb = _sincos_block([rel[:, 0:1], rel[:, 1:2], rel[:, 2:3], d], _HID, 32)
    h = jax.nn.gelu(
        lax.dot_general(emb, w1_ref[...], (((1,), (0,)), ((), ()))) + b1_ref[...])
    msg = lax.dot_general(h, w2_ref[...], (((1,), (0,)), ((), ()))) + b2_ref[...]

    rp = lax.broadcasted_iota(jnp.int32, (BSUP, R), 1)
    sp = lax.broadcasted_iota(jnp.int32, (BSUP, R), 0)
    pool = jnp.where(rp // _K == sp, jnp.float32(1.0 / _K), jnp.float32(0.0))
    agg = lax.dot_general(pool, msg, (((1,), (0,)), ((), ())), precision=hi)

    spe = _sincos_block([sup[:, 0:1], sup[:, 1:2], sup[:, 2:3]], _HID, 42)
    out = (lax.dot_general(agg, wp_ref[0:_HID, :], (((1,), (0,)), ((), ())))
           + lax.dot_general(spe, wp_ref[_HID:2 * _HID, :],
                             (((1,), (0,)), ((), ())))
           + bp_ref[...])
    out_ref[0] = out


def _mlp(nbr16, sup16, W1, b1, W2, b2, Wp, bp):
    grid = (_S * _K) // _MLP_ROWS
    full = lambda i: (0, 0)
    return pl.pallas_call(
        _mlp_body,
        grid=(grid,),
        in_specs=[
            pl.BlockSpec((_MLP_ROWS, _PAD_D), lambda i: (i, 0)),
            pl.BlockSpec((_MLP_SUP, _PAD_D), lambda i: (i, 0)),
            pl.BlockSpec((_HID, _HID), full),
            pl.BlockSpec((1, _HID), full),
            pl.BlockSpec((_HID, _HID), full),
            pl.BlockSpec((1, _HID), full),
            pl.BlockSpec((2 * _HID, _HID), full),
            pl.BlockSpec((1, _HID), full),
        ],
        out_specs=pl.BlockSpec((1, _MLP_SUP, _HID), lambda i: (0, i, 0)),
        out_shape=jax.ShapeDtypeStruct((1, _S, _HID), jnp.float32),
    )(nbr16, sup16, W1, b1, W2, b2, Wp, bp)


def kernel(positions, supernode_indices, W1, b1, W2, b2, Wp, bp):
    pos_pad = jnp.pad(positions, ((0, 0), (0, _PAD_D - 3)))
    sidx = supernode_indices.astype(jnp.int32)
    sup16 = _sc_gather_rows(pos_pad, sidx)
    pos_t = positions.T
    nbr_idx = _topk(pos_t, sup16)
    nbr16 = _sc_gather_rows(pos_pad, nbr_idx.reshape(-1))
    return _mlp(nbr16, sup16, W1, b1.reshape(1, -1), W2, b2.reshape(1, -1),
                Wp, bp.reshape(1, -1))


# speedup vs baseline: 3.2603x; 1.0148x over previous
import functools

import jax
import jax.numpy as jnp
from jax import lax
from jax.experimental import pallas as pl
from jax.experimental.pallas import tpu as pltpu
from jax.experimental.pallas import tpu_sc as plsc

_HID = 256
_K = 16
_N = 32768
_S = 1024
_PAD_D = 16

_TOPK_BS = 64
_TOPK_C = 2048
_TOPK_CH = 512
_MLP_ROWS = 2048
_MLP_SUP = _MLP_ROWS // _K

_LOG1E4 = 9.210340371976184


def _sc_gather_rows(table, idx):
    B = idx.shape[0]
    info = plsc.get_sparse_core_info()
    nw = info.num_cores * info.num_subcores
    bpw = B // nw
    mesh = plsc.VectorSubcoreMesh(core_axis_name="c", subcore_axis_name="s")

    @functools.partial(
        pl.kernel,
        mesh=mesh,
        compiler_params=pltpu.CompilerParams(use_tc_tiling_on_sc=False),
        out_type=jax.ShapeDtypeStruct((B, _PAD_D), jnp.float32),
        scratch_types=[
            pltpu.VMEM((bpw,), jnp.int32),
            pltpu.VMEM((bpw, _PAD_D), jnp.float32),
            pltpu.SemaphoreType.DMA,
        ],
    )
    def gather_kernel(table_hbm, idx_hbm, out_hbm, idx_v, rows_v, sem):
        wid = lax.axis_index("s") * info.num_cores + lax.axis_index("c")
        base = wid * bpw
        pltpu.sync_copy(idx_hbm.at[pl.ds(base, bpw)], idx_v)
        pltpu.async_copy(table_hbm.at[idx_v], rows_v, sem).wait()
        pltpu.sync_copy(rows_v, out_hbm.at[pl.ds(base, bpw)])

    return gather_kernel(table, idx)


def _topk_body(pos_ref, sup_ref, out_ref, d2_ref):
    n_chunks = _N // _TOPK_C
    s0 = sup_ref[:, 0:1]
    s1 = sup_ref[:, 1:2]
    s2 = sup_ref[:, 2:3]

    bl = lax.broadcasted_iota(jnp.int32, (_TOPK_BS, _TOPK_C), 1)

    def build(c, carry):
        sl = pl.ds(c * _TOPK_C, _TOPK_C)
        p0 = pos_ref[0:1, sl]
        p1 = pos_ref[1:2, sl]
        p2 = pos_ref[2:3, sl]
        t0 = s0 - p0
        t1 = s1 - p1
        t2 = s2 - p2
        d2 = (t0 * t0 + t1 * t1) + t2 * t2
        cid = jax.lax.shift_right_logical(bl, 9) + (c * (_TOPK_C // _TOPK_CH))
        bits = jax.lax.bitcast_convert_type(d2 + 1.0, jnp.int32)
        keys = jax.lax.bitwise_or(jax.lax.bitwise_and(bits, -128), cid)
        d2_ref[:, sl] = jax.lax.bitcast_convert_type(keys, jnp.float32)
        return carry

    lax.fori_loop(0, n_chunks, build, 0)

    lane_k = lax.broadcasted_iota(jnp.int32, (_TOPK_BS, _K), 1)
    big = jnp.float32(jnp.inf)
    ch = _TOPK_CH
    n_ch2 = _N // ch
    lane_ch = lax.broadcasted_iota(jnp.int32, (_TOPK_BS, ch), 1)

    def extract(t, carry):
        pi, acc = carry

        def scan_chunk(c4, av):
            for u in range(8):
                c = c4 * 8 + u
                sl = pl.ds(c * ch, ch)
                dc = d2_ref[:, sl]
                dcm = jnp.where(lane_ch == (pi - c * ch)[:, None], big, dc)
                d2_ref[:, sl] = dcm
                av = jnp.minimum(av, dcm)
            return av

        av0 = jnp.full((_TOPK_BS, ch), big, jnp.float32)
        av = lax.fori_loop(0, n_ch2 // 8, scan_chunk, av0)
        mv = jnp.min(av, axis=1)
        li = jnp.min(jnp.where(av == mv[:, None], lane_ch, _N), axis=1)
        cw = jax.lax.bitwise_and(
            jax.lax.bitcast_convert_type(mv, jnp.int32), 127)
        bi = cw * ch + li
        acc = jnp.where(lane_k == t, bi[:, None], acc)
        return bi, acc

    pi0 = jnp.full((_TOPK_BS,), -1, jnp.int32)
    acc0 = jnp.zeros((_TOPK_BS, _K), jnp.int32)
    _, idxs = lax.fori_loop(0, _K, extract, (pi0, acc0))
    out_ref[...] = idxs


def _topk(pos_t, sup16):
    grid = _S // _TOPK_BS
    return pl.pallas_call(
        _topk_body,
        grid=(grid,),
        in_specs=[
            pl.BlockSpec((3, _N), lambda i: (0, 0)),
            pl.BlockSpec((_TOPK_BS, _PAD_D), lambda i: (i, 0)),
        ],
        out_specs=pl.BlockSpec((_TOPK_BS, _K), lambda i: (i, 0)),
        out_shape=jax.ShapeDtypeStruct((_S, _K), jnp.int32),
        scratch_shapes=[pltpu.VMEM((_TOPK_BS, _N), jnp.float32)],
    )(pos_t, sup16)


def _sincos_block(vals, widths, half):
    R = vals[0].shape[0]
    nb = len(vals)
    fb = jnp.concatenate(
        [jnp.broadcast_to(v, (R, half)) for v in vals], axis=1)
    ji = lax.broadcasted_iota(jnp.int32, (1, nb * half), 1)
    jm = (ji % half).astype(jnp.float32)
    om = 1.0 / jnp.exp((jm / half) * _LOG1E4)
    ang = fb * om
    sn = jnp.sin(ang)
    cs = jnp.cos(ang)
    parts = []
    for c in range(nb):
        parts.append(sn[:, c * half:(c + 1) * half])
        parts.append(cs[:, c * half:(c + 1) * half])
    if widths > 2 * half * nb:
        parts.append(jnp.zeros((R, widths - 2 * half * nb), jnp.float32))
    return jnp.concatenate(parts, axis=1)


def _mlp_body(nbr_ref, sup_ref, w1_ref, b1_ref, w2_ref, b2_ref, wp_ref,
              bp_ref, out_ref):
    R, BSUP = _MLP_ROWS, _MLP_SUP
    hi = lax.Precision.HIGHEST
    nbr = nbr_ref[...]
    sup = sup_ref[...]

    r_iota = lax.broadcasted_iota(jnp.int32, (R, BSUP), 0)
    s_iota = lax.broadcasted_iota(jnp.int32, (R, BSUP), 1)
    expand = (r_iota // _K == s_iota).astype(jnp.float32)
    sup_rows = lax.dot_general(expand, sup, (((1,), (0,)), ((), ())),
                               precision=hi)
    rel = nbr - sup_rows
    rsq = rel * rel
    d2 = (rsq[:, 0:1] + rsq[:, 1:2]) + rsq[:, 2:3]
    d = jnp.sqrt(d2 + 1e-12)

    emb = _sincos_block([rel[:, 0:1], rel[:, 1:2], rel[:, 2:3], d], _HID, 32)
    h = jax.nn.gelu(
        lax.dot_general(emb, w1_ref[...], (((1,), (0,)), ((), ()))) + b1_ref[...])
    msg = lax.dot_general(h, w2_ref[...], (((1,), (0,)), ((), ()))) + b2_ref[...]

    rp = lax.broadcasted_iota(jnp.int32, (BSUP, R), 1)
    sp = lax.broadcasted_iota(jnp.int32, (BSUP, R), 0)
    pool = jnp.where(rp // _K == sp, jnp.float32(1.0 / _K), jnp.float32(0.0))
    agg = lax.dot_general(pool, msg, (((1,), (0,)), ((), ())), precision=hi)

    spe = _sincos_block([sup[:, 0:1], sup[:, 1:2], sup[:, 2:3]], _HID, 42)
    out = (lax.dot_general(agg, wp_ref[0:_HID, :], (((1,), (0,)), ((), ())))
           + lax.dot_general(spe, wp_ref[_HID:2 * _HID, :],
                             (((1,), (0,)), ((), ())))
           + bp_ref[...])
    out_ref[0] = out


def _mlp(nbr16, sup16, W1, b1, W2, b2, Wp, bp):
    grid = (_S * _K) // _MLP_ROWS
    full = lambda i: (0, 0)
    return pl.pallas_call(
        _mlp_body,
        grid=(grid,),
        in_specs=[
            pl.BlockSpec((_MLP_ROWS, _PAD_D), lambda i: (i, 0)),
            pl.BlockSpec((_MLP_SUP, _PAD_D), lambda i: (i, 0)),
            pl.BlockSpec((_HID, _HID), full),
            pl.BlockSpec((1, _HID), full),
            pl.BlockSpec((_HID, _HID), full),
            pl.BlockSpec((1, _HID), full),
            pl.BlockSpec((2 * _HID, _HID), full),
            pl.BlockSpec((1, _HID), full),
        ],
        out_specs=pl.BlockSpec((1, _MLP_SUP, _HID), lambda i: (0, i, 0)),
        out_shape=jax.ShapeDtypeStruct((1, _S, _HID), jnp.float32),
    )(nbr16, sup16, W1, b1, W2, b2, Wp, bp)


def kernel(positions, supernode_indices, W1, b1, W2, b2, Wp, bp):
    pos_pad = jnp.pad(positions, ((0, 0), (0, _PAD_D - 3)))
    sidx = supernode_indices.astype(jnp.int32)
    sup16 = _sc_gather_rows(pos_pad, sidx)
    pos_t = positions.T
    nbr_idx = _topk(pos_t, sup16)
    nbr16 = _sc_gather_rows(pos_pad, nbr_idx.reshape(-1))
    return _mlp(nbr16, sup16, W1, b1.reshape(1, -1), W2, b2.reshape(1, -1),
                Wp, bp.reshape(1, -1))
